# baseline (speedup 1.0000x reference)
import functools

import jax
import jax.numpy as jnp
from jax import lax
from jax.experimental import pallas as pl
from jax.experimental.pallas import tpu as pltpu
from jax.experimental.pallas import tpu_sc as plsc

UST_DIM = 8
_NC, _NS, _LANES = 2, 16, 16
_NW = _NC * _NS


def _make_sc_scale(F):
    q_per_w = F // _NW
    n_qv = q_per_w // _LANES
    mesh = plsc.VectorSubcoreMesh(core_axis_name="c", subcore_axis_name="s")

    @functools.partial(
        pl.kernel,
        mesh=mesh,
        out_type=jax.ShapeDtypeStruct((F,), jnp.float32),
        scratch_types=[pltpu.VMEM((q_per_w,), jnp.float32)],
    )
    def scale_sc(out_hbm, buf_v):
        wid = lax.axis_index("s") * _NC + lax.axis_index("c")
        base = wid * q_per_w
        lane = lax.iota(jnp.int32, _LANES)
        for qv in range(n_qv):
            qf = (base + qv * _LANES + lane).astype(jnp.float32)

            def nbody(step, carry, qf=qf):
                mind, mini = carry
                for u in range(8):
                    n = step * 8 + u
                    diff = qf - n.astype(jnp.float32)
                    dist = jnp.float32(UST_DIM) * (diff * diff)
                    better = dist < mind
                    mind = jnp.where(better, dist, mind)
                    mini = jnp.where(better, n, mini)
                return mind, mini

            mind0 = jnp.full((_LANES,), jnp.float32(3.4e38))
            mini0 = jnp.zeros((_LANES,), jnp.int32)
            _, mini = lax.fori_loop(0, F // 8, nbody, (mind0, mini0))
            buf_v[pl.ds(qv * _LANES, _LANES)] = (
                mini.astype(jnp.float32) + 1.0
            ) / jnp.float32(F)
        pltpu.sync_copy(buf_v, out_hbm.at[pl.ds(base, q_per_w)])

    return scale_sc


def _mul_kernel(x_ref, scale_ref, o_ref):
    o_ref[...] = x_ref[...] * scale_ref[...]


def kernel(inputs):
    B, F = inputs.shape
    scale = _make_sc_scale(F)()
    BLK = 2048
    out = pl.pallas_call(
        _mul_kernel,
        grid=(B // BLK,),
        in_specs=[
            pl.BlockSpec((BLK, F), lambda i: (i, 0)),
            pl.BlockSpec((1, F), lambda i: (0, 0)),
        ],
        out_specs=pl.BlockSpec((BLK, F), lambda i: (i, 0)),
        out_shape=jax.ShapeDtypeStruct((B, F), inputs.dtype),
        compiler_params=pltpu.CompilerParams(
            dimension_semantics=("arbitrary",),
        ),
    )(inputs, scale.reshape(1, F))
    return out
